# baseline (speedup 1.0000x reference)
import functools

import jax
import jax.numpy as jnp
from jax import lax
from jax.experimental import pallas as pl
from jax.experimental.pallas import tpu as pltpu
from jax.experimental.pallas import tpu_sc as plsc

D = 16
NC, NS = 2, 16
NW = NC * NS
B_TOTAL = 16384 * 50
B_PER_W = B_TOTAL // NW
CHUNK = 3200
N_CHUNKS = B_PER_W // CHUNK


def _emb_body(idx_hbm, table_hbm, out_hbm, idx_v, rows_v, sem):
    wid = lax.axis_index("s") * NC + lax.axis_index("c")
    base = wid * B_PER_W
    for i in range(N_CHUNKS):
        off = base + i * CHUNK
        pltpu.sync_copy(idx_hbm.at[pl.ds(off, CHUNK)], idx_v)
        pltpu.async_copy(table_hbm.at[idx_v], rows_v, sem).wait()
        pltpu.sync_copy(rows_v, out_hbm.at[pl.ds(off, CHUNK)])


@jax.jit
def _embed(idx_flat, W):
    mesh = plsc.VectorSubcoreMesh(core_axis_name="c", subcore_axis_name="s")
    fn = functools.partial(
        pl.kernel,
        mesh=mesh,
        out_type=jax.ShapeDtypeStruct((B_TOTAL, D), jnp.float32),
        scratch_types=[
            pltpu.VMEM((CHUNK,), jnp.int32),
            pltpu.VMEM((CHUNK, D), jnp.float32),
            pltpu.SemaphoreType.DMA,
        ],
        compiler_params=pltpu.CompilerParams(use_tc_tiling_on_sc=False),
    )(_emb_body)
    return fn(idx_flat, W)


def kernel(inputs, W):
    out = _embed(inputs.reshape(-1), W)
    return out.reshape(inputs.shape[0], inputs.shape[1], D)


# speedup vs baseline: 1.0101x; 1.0101x over previous
import functools

import jax
import jax.numpy as jnp
from jax import lax
from jax.experimental import pallas as pl
from jax.experimental.pallas import tpu as pltpu
from jax.experimental.pallas import tpu_sc as plsc

D = 16
NC, NS = 2, 16
NW = NC * NS
B_TOTAL = 16384 * 50
B_PER_W = B_TOTAL // NW
CHUNK = 3200
N_CHUNKS = B_PER_W // CHUNK


def _emb_body(idx_hbm, table_hbm, out_hbm, idx_v, rows0, rows1,
              gsem0, gsem1, wsem0, wsem1):
    wid = lax.axis_index("s") * NC + lax.axis_index("c")
    base = wid * B_PER_W
    rows = (rows0, rows1)
    gsems = (gsem0, gsem1)
    wsems = (wsem0, wsem1)
    pltpu.sync_copy(idx_hbm.at[pl.ds(base, B_PER_W)], idx_v)
    gath = [None] * N_CHUNKS
    wb = [None] * N_CHUNKS
    gath[0] = pltpu.async_copy(
        table_hbm.at[idx_v.at[pl.ds(0, CHUNK)]], rows[0], gsems[0])
    for i in range(N_CHUNKS):
        if i + 1 < N_CHUNKS:
            if i >= 1:
                wb[i - 1].wait()
            gath[i + 1] = pltpu.async_copy(
                table_hbm.at[idx_v.at[pl.ds((i + 1) * CHUNK, CHUNK)]],
                rows[(i + 1) % 2], gsems[(i + 1) % 2])
        gath[i].wait()
        wb[i] = pltpu.async_copy(
            rows[i % 2], out_hbm.at[pl.ds(base + i * CHUNK, CHUNK)],
            wsems[i % 2])
    wb[N_CHUNKS - 2].wait()
    wb[N_CHUNKS - 1].wait()


@jax.jit
def _embed(idx_flat, W):
    mesh = plsc.VectorSubcoreMesh(core_axis_name="c", subcore_axis_name="s")
    fn = functools.partial(
        pl.kernel,
        mesh=mesh,
        out_type=jax.ShapeDtypeStruct((B_TOTAL, D), jnp.float32),
        scratch_types=[
            pltpu.VMEM((B_PER_W,), jnp.int32),
            pltpu.VMEM((CHUNK, D), jnp.float32),
            pltpu.VMEM((CHUNK, D), jnp.float32),
            pltpu.SemaphoreType.DMA,
            pltpu.SemaphoreType.DMA,
            pltpu.SemaphoreType.DMA,
            pltpu.SemaphoreType.DMA,
        ],
        compiler_params=pltpu.CompilerParams(use_tc_tiling_on_sc=False),
    )(_emb_body)
    return fn(idx_flat, W)


def kernel(inputs, W):
    out = _embed(inputs.reshape(-1), W)
    return out.reshape(inputs.shape[0], inputs.shape[1], D)


# speedup vs baseline: 1.2783x; 1.2655x over previous
import functools

import jax
import jax.numpy as jnp
from jax import lax
from jax.experimental import pallas as pl
from jax.experimental.pallas import tpu as pltpu
from jax.experimental.pallas import tpu_sc as plsc

D = 16
S = 50
NC, NS = 2, 16
NW = NC * NS
R_TOTAL = 16384
R_PER_W = R_TOTAL // NW
R_CHUNK = 64
CHUNK = R_CHUNK * S
B_PER_W = R_PER_W * S
N_CHUNKS = R_PER_W // R_CHUNK


def _emb_body(idx_hbm, table_hbm, out_hbm, idx_v, rows0, rows1,
              gsem0, gsem1, wsem0, wsem1):
    wid = lax.axis_index("s") * NC + lax.axis_index("c")
    base = wid * B_PER_W
    row0 = wid * R_PER_W
    rows = (rows0, rows1)
    gsems = (gsem0, gsem1)
    wsems = (wsem0, wsem1)
    pltpu.sync_copy(idx_hbm.at[pl.ds(base, B_PER_W)], idx_v)
    gath = [None] * N_CHUNKS
    wb = [None] * N_CHUNKS
    gath[0] = pltpu.async_copy(
        table_hbm.at[idx_v.at[pl.ds(0, CHUNK)]], rows[0], gsems[0])
    for i in range(N_CHUNKS):
        if i + 1 < N_CHUNKS:
            if i >= 1:
                for h in wb[i - 1]:
                    h.wait()
            gath[i + 1] = pltpu.async_copy(
                table_hbm.at[idx_v.at[pl.ds((i + 1) * CHUNK, CHUNK)]],
                rows[(i + 1) % 2], gsems[(i + 1) % 2])
        gath[i].wait()
        wb[i] = [
            pltpu.async_copy(
                rows[i % 2].at[pl.ds(j * S, S), :],
                out_hbm.at[row0 + i * R_CHUNK + j, :, :],
                wsems[i % 2])
            for j in range(R_CHUNK)
        ]
    for h in wb[N_CHUNKS - 2]:
        h.wait()
    for h in wb[N_CHUNKS - 1]:
        h.wait()


@jax.jit
def _embed(idx_flat, W):
    mesh = plsc.VectorSubcoreMesh(core_axis_name="c", subcore_axis_name="s")
    fn = functools.partial(
        pl.kernel,
        mesh=mesh,
        out_type=jax.ShapeDtypeStruct((R_TOTAL, S, D), jnp.float32),
        scratch_types=[
            pltpu.VMEM((B_PER_W,), jnp.int32),
            pltpu.VMEM((CHUNK, D), jnp.float32),
            pltpu.VMEM((CHUNK, D), jnp.float32),
            pltpu.SemaphoreType.DMA,
            pltpu.SemaphoreType.DMA,
            pltpu.SemaphoreType.DMA,
            pltpu.SemaphoreType.DMA,
        ],
        compiler_params=pltpu.CompilerParams(use_tc_tiling_on_sc=False),
    )(_emb_body)
    return fn(idx_flat, W)


def kernel(inputs, W):
    return _embed(inputs.reshape(-1), W)


# speedup vs baseline: 1.2801x; 1.0014x over previous
import functools

import jax
import jax.numpy as jnp
from jax import lax
from jax.experimental import pallas as pl
from jax.experimental.pallas import tpu as pltpu
from jax.experimental.pallas import tpu_sc as plsc

D = 16
S = 50
NC, NS = 2, 16
NW = NC * NS
R_TOTAL = 16384
R_PER_W = R_TOTAL // NW
R_CHUNK = 64
CHUNK = R_CHUNK * S
B_PER_W = R_PER_W * S
N_CHUNKS = R_PER_W // R_CHUNK


def _emb_body(idx_hbm, table_hbm, out_hbm, idx_v, rows0, rows1,
              gsem0, gsem1, wsem0, wsem1):
    wid = lax.axis_index("s") * NC + lax.axis_index("c")
    base = wid * B_PER_W
    row0 = wid * R_PER_W
    rows = (rows0, rows1)
    gsems = (gsem0, gsem1)
    wsems = (wsem0, wsem1)
    pltpu.sync_copy(idx_hbm.at[pl.ds(base, B_PER_W)], idx_v)
    gath = [None] * N_CHUNKS
    wb = [None] * N_CHUNKS
    gath[0] = pltpu.async_copy(
        table_hbm.at[idx_v.at[pl.ds(0, CHUNK)]], rows[0], gsems[0])
    for i in range(N_CHUNKS):
        if i + 1 < N_CHUNKS:
            if i >= 1:
                for h in wb[i - 1]:
                    h.wait()
            gath[i + 1] = pltpu.async_copy(
                table_hbm.at[idx_v.at[pl.ds((i + 1) * CHUNK, CHUNK)]],
                rows[(i + 1) % 2], gsems[(i + 1) % 2])
        gath[i].wait()
        wb[i] = [
            pltpu.async_copy(
                rows[i % 2].at[pl.ds(j * S, S), :],
                out_hbm.at[row0 + i * R_CHUNK + j, :, :],
                wsems[i % 2])
            for j in range(R_CHUNK)
        ]
    for h in wb[N_CHUNKS - 2]:
        h.wait()
    for h in wb[N_CHUNKS - 1]:
        h.wait()


@jax.jit
def _embed(idx_flat, W):
    mesh = plsc.VectorSubcoreMesh(core_axis_name="c", subcore_axis_name="s")
    fn = functools.partial(
        pl.kernel,
        mesh=mesh,
        out_type=jax.ShapeDtypeStruct((R_TOTAL, S, D), jnp.float32),
        scratch_types=[
            pltpu.VMEM((B_PER_W,), jnp.int32),
            pltpu.VMEM((CHUNK, D), jnp.float32),
            pltpu.VMEM((CHUNK, D), jnp.float32),
            pltpu.SemaphoreType.DMA,
            pltpu.SemaphoreType.DMA,
            pltpu.SemaphoreType.DMA,
            pltpu.SemaphoreType.DMA,
        ],
        compiler_params=pltpu.CompilerParams(use_tc_tiling_on_sc=False),
    )(_emb_body)
    return fn(idx_flat, W)


def kernel(inputs, W):
    idx_flat = jnp.minimum(inputs.reshape(-1), W.shape[0] - 1)
    return _embed(idx_flat, W)
